# baseline (speedup 1.0000x reference)
import functools

import jax
import jax.numpy as jnp
from jax.experimental import pallas as pl
from jax.experimental.pallas import tpu as pltpu
from jax.experimental.pallas import tpu_sc as plsc

F32 = jnp.float32
HI = jax.lax.Precision.HIGHEST

NC, NS, L = 2, 16, 16
NW = NC * NS
CH = 128


def _sig(x):
    return 1.0 / (1.0 + jnp.exp(-x))


def _tanh(x):
    a = jnp.abs(x)
    e = jnp.exp(-2.0 * a)
    return jnp.sign(x) * (1.0 - e) / (1.0 + e)


def _k1_body(nf_ref, w_ref, p_ref, q_ref, nfb_ref):
    y = jnp.dot(nf_ref[...], w_ref[...], preferred_element_type=F32,
                precision=HI)
    p_ref[...] = y[:, 0:16]
    q_ref[...] = y[:, 16:32]
    nfb_ref[...] = y[:, 32:160]


def _k1(nf, w_all):
    n = nf.shape[0]
    blk = 2000
    grid = n // blk
    return pl.pallas_call(
        _k1_body,
        grid=(grid,),
        in_specs=[
            pl.BlockSpec((blk, 128), lambda i: (i, 0)),
            pl.BlockSpec((128, 160), lambda i: (0, 0)),
        ],
        out_specs=(
            pl.BlockSpec((blk, 16), lambda i: (i, 0)),
            pl.BlockSpec((blk, 16), lambda i: (i, 0)),
            pl.BlockSpec((blk, 128), lambda i: (i, 0)),
        ),
        out_shape=(
            jax.ShapeDtypeStruct((n, 16), F32),
            jax.ShapeDtypeStruct((n, 16), F32),
            jax.ShapeDtypeStruct((n, 128), F32),
        ),
    )(nf, w_all)


def _k2_body(x_ref, ab_ref, gf_ref, agf_ref, be_ref, t16_ref, out_ref):
    ce = jnp.dot(gf_ref[...], agf_ref[...], preferred_element_type=F32,
                 precision=HI) + be_ref[...]
    ce128 = jnp.dot(ce, t16_ref[...], preferred_element_type=F32,
                    precision=HI)
    out_ref[...] = jnp.dot(x_ref[...], ab_ref[...],
                           preferred_element_type=F32, precision=HI) + ce128


def _k2(ef8, a_blk, gf, a_gf, be, t16):
    e8 = ef8.shape[0]
    blk = 8000
    grid = e8 // blk
    return pl.pallas_call(
        _k2_body,
        grid=(grid,),
        in_specs=[
            pl.BlockSpec((blk, 128), lambda i: (i, 0)),
            pl.BlockSpec((128, 128), lambda i: (0, 0)),
            pl.BlockSpec((1, 32), lambda i: (0, 0)),
            pl.BlockSpec((32, 16), lambda i: (0, 0)),
            pl.BlockSpec((1, 16), lambda i: (0, 0)),
            pl.BlockSpec((16, 128), lambda i: (0, 0)),
        ],
        out_specs=pl.BlockSpec((blk, 128), lambda i: (i, 0)),
        out_shape=jax.ShapeDtypeStruct((e8, 128), F32),
    )(ef8, a_blk, gf, a_gf, be, t16)


def _sc_edge(src, dst, ef8, efa8, p_tab, q_tab):
    e = ef8.shape[0] * 8
    n = p_tab.shape[0]
    nchunks = e // CH
    base_c, extra = nchunks // NW, nchunks % NW
    nrows_t = n // NS
    CR = CH // 8

    mesh = plsc.VectorSubcoreMesh(core_axis_name="c", subcore_axis_name="s")

    @functools.partial(
        pl.kernel,
        out_type=(
            jax.ShapeDtypeStruct((e // 8, 128), F32),
            jax.ShapeDtypeStruct((NC, NS, nrows_t, 16), F32),
            jax.ShapeDtypeStruct((NC, NS, n), F32),
            jax.ShapeDtypeStruct((NW, 1, 16), F32),
        ),
        mesh=mesh,
        compiler_params=pltpu.CompilerParams(needs_layout_passes=False,
                                             use_tc_tiling_on_sc=False),
        scratch_types=dict(
            src_v=pltpu.VMEM((CH,), jnp.int32),
            dst_v=pltpu.VMEM((CH,), jnp.int32),
            ef_v=pltpu.VMEM((CR, 128), F32),
            efa_v=pltpu.VMEM((CR, 128), F32),
            xe_v=pltpu.VMEM((CR, 128), F32),
            pr_v=pltpu.VMEM((CH, 16), F32),
            qr_v=pltpu.VMEM((CH, 16), F32),
            efn_v=pltpu.VMEM((CH, 16), F32),
            deg_v=pltpu.VMEM((n,), F32),
            sum_v=pltpu.VMEM((16,), F32),
            zrow_v=pltpu.VMEM((n // NS, 16), F32),
            acc_sh=pltpu.VMEM_SHARED((n, 16), F32),
            sem_p=pltpu.SemaphoreType.DMA,
            sem_q=pltpu.SemaphoreType.DMA,
        ),
    )
    def k(src_h, dst_h, ef_h, efa_h, p_h, q_h, xe_h, acc_h, deg_h, sums_h,
          src_v, dst_v, ef_v, efa_v, xe_v, pr_v, qr_v, efn_v, deg_v, sum_v,
          zrow_v, acc_sh, sem_p, sem_q):
        cid = jax.lax.axis_index("c")
        sid = jax.lax.axis_index("s")
        wid = sid * NC + cid

        zf = jnp.zeros((L,), F32)

        def zloop(i, _):
            deg_v[pl.ds(i * L, L)] = zf
            return 0
        jax.lax.fori_loop(0, n // L, zloop, 0)

        def zrow(i, _):
            zrow_v[i, :] = zf
            return 0
        jax.lax.fori_loop(0, nrows_t, zrow, 0)

        pltpu.sync_copy(zrow_v, acc_sh.at[pl.ds(sid * nrows_t, nrows_t)])
        plsc.subcore_barrier()

        ones = jnp.ones((L,), F32)
        my_chunks = base_c + jnp.where(wid < extra, 1, 0)
        chunk0 = wid * base_c + jnp.minimum(wid, extra)

        def chunk_body(i, acc_sum):
            c = chunk0 + i
            base = c * CH
            pltpu.sync_copy(src_h.at[pl.ds(base, CH)], src_v)
            pltpu.sync_copy(dst_h.at[pl.ds(base, CH)], dst_v)
            cp = pltpu.async_copy(p_h.at[src_v], pr_v, sem_p)
            cq = pltpu.async_copy(q_h.at[dst_v], qr_v, sem_q)
            pltpu.sync_copy(ef_h.at[pl.ds(c * CR, CR)], ef_v)
            pltpu.sync_copy(efa_h.at[pl.ds(c * CR, CR)], efa_v)
            cp.wait()
            cq.wait()

            def row_body(r, s):
                g = jax.lax.shift_right_logical(r, 3)
                o = jax.lax.bitwise_and(r, 7) * 16
                efn = efa_v[g, pl.ds(o, 16)] + pr_v[r, :] + qr_v[r, :]
                efn_v[r, :] = efn
                xe_v[g, pl.ds(o, 16)] = ef_v[g, pl.ds(o, 16)] + efn
                return s + efn
            acc_sum = jax.lax.fori_loop(0, CH, row_body, acc_sum)

            def deg_body(r, _):
                idx = src_v[pl.ds(r * L, L)]
                plsc.addupdate_scatter(deg_v, [idx], ones)
                return 0
            jax.lax.fori_loop(0, CH // L, deg_body, 0)

            pltpu.sync_copy(efn_v, acc_sh.at[src_v], add=True)
            pltpu.sync_copy(xe_v, xe_h.at[pl.ds(c * CR, CR)])
            return acc_sum

        total = jax.lax.fori_loop(0, my_chunks, chunk_body,
                                  jnp.zeros((16,), F32))
        sum_v[...] = total

        pltpu.sync_copy(sum_v, sums_h.at[wid, 0])
        pltpu.sync_copy(deg_v, deg_h.at[cid, sid])
        plsc.subcore_barrier()
        pltpu.sync_copy(acc_sh.at[pl.ds(sid * nrows_t, nrows_t)],
                        acc_h.at[cid, sid])

    return k(src, dst, ef8, efa8, p_tab, q_tab)


def _k3_body(nf_ref, nfb_ref, acc_ref, deg_ref, bem_ref, gf_ref, bgf_ref,
             bn_ref, xn_ref, ngs_ref):
    acc = acc_ref[0] + acc_ref[1]
    deg = jnp.sum(deg_ref[...], axis=(0, 1))
    emean = acc / jnp.maximum(deg, 1.0)[:, None]
    cn = jnp.dot(gf_ref[...], bgf_ref[...], preferred_element_type=F32,
                 precision=HI) + bn_ref[...]
    nf_n = nfb_ref[...] + jnp.dot(emean, bem_ref[...],
                                  preferred_element_type=F32,
                                  precision=HI) + cn
    xn_ref[...] = nf_ref[...] + nf_n
    ngs_ref[...] = jnp.sum(nf_n, axis=0, keepdims=True)


def _k3(nf, nfb, acc, degp, b_em, gf, b_gf, bn):
    n = nf.shape[0]
    return pl.pallas_call(
        _k3_body,
        out_shape=(
            jax.ShapeDtypeStruct((n, 128), F32),
            jax.ShapeDtypeStruct((1, 128), F32),
        ),
    )(nf, nfb, acc, degp, b_em, gf, b_gf, bn)


def _s2s_lstm(dd, h, r, c, wih, wil, wh, bsum):
    gates = (jnp.dot(h, wih, preferred_element_type=F32, precision=HI)
             + jnp.dot(r, wil, preferred_element_type=F32, precision=HI)
             + jnp.dot(h, wh, preferred_element_type=F32, precision=HI)
             + bsum)
    ig = gates[:, 0:dd]
    fg = gates[:, dd:2 * dd]
    gg = gates[:, 2 * dd:3 * dd]
    og = gates[:, 3 * dd:4 * dd]
    c = _sig(fg) * c + _sig(ig) * _tanh(gg)
    h = _sig(og) * _tanh(c)
    return h, c


def _s2s_node_body(x_ref, wih_ref, wil_ref, wh_ref, bsum_ref, q1_ref, c1_ref,
                   out_ref):
    x = x_ref[...]

    def attend(q):
        e = jnp.sum(x * q, axis=1, keepdims=True)
        m = jnp.max(e)
        a = jnp.exp(e - m)
        s = jnp.sum(a)
        return jnp.sum(a * x, axis=0, keepdims=True) / s

    h = q1_ref[...]
    c = c1_ref[...]
    r = attend(h)
    for _ in range(2):
        h, c = _s2s_lstm(128, h, r, c, wih_ref[...], wil_ref[...],
                         wh_ref[...], bsum_ref[...])
        r = attend(h)
    out_ref[:, 0:128] = h
    out_ref[:, 128:256] = r


def _s2s_edge_body(nchunks, chk, x_ref, wih_ref, wil_ref, wh_ref, bsum_ref,
                   q1_ref, c1_ref, t16_ref, m8_ref, k8_ref, t16t_ref,
                   out_ref):
    m8 = m8_ref[...]
    k8 = k8_ref[...]

    def attend(q):
        tq = jnp.dot(q, t16_ref[...], preferred_element_type=F32,
                     precision=HI)
        m = jnp.full((1, 8), -3.4e38, F32)
        s = jnp.zeros((1, 8), F32)
        r128 = jnp.zeros((1, 128), F32)
        for i in range(nchunks):
            xc = x_ref[pl.ds(i * chk, chk), :]
            eg = jnp.dot(xc * tq, m8, preferred_element_type=F32,
                         precision=HI)
            mn = jnp.maximum(m, jnp.max(eg, axis=0, keepdims=True))
            alpha = jnp.exp(m - mn)
            a = jnp.exp(eg - mn)
            s = s * alpha + jnp.sum(a, axis=0, keepdims=True)
            aw = jnp.dot(a, k8, preferred_element_type=F32, precision=HI)
            al128 = jnp.dot(alpha, k8, preferred_element_type=F32,
                            precision=HI)
            r128 = r128 * al128 + jnp.sum(xc * aw, axis=0, keepdims=True)
            m = mn
        mg = jnp.max(m)
        w = jnp.exp(m - mg)
        sg = jnp.sum(s * w)
        w128 = jnp.dot(w, k8, preferred_element_type=F32, precision=HI)
        return jnp.dot(r128 * w128, t16t_ref[...],
                       preferred_element_type=F32, precision=HI) / sg

    h = q1_ref[...]
    c = c1_ref[...]
    r = attend(h)
    for _ in range(2):
        h, c = _s2s_lstm(16, h, r, c, wih_ref[...], wil_ref[...],
                         wh_ref[...], bsum_ref[...])
        r = attend(h)
    out_ref[:, 0:16] = h
    out_ref[:, 16:32] = r


def _s2s_consts(wi, wh, bi, bh, dd):
    bsum = (bi + bh)[None, :]
    g1 = bi + bh
    i1 = g1[0:dd]
    g1g = g1[2 * dd:3 * dd]
    o1 = g1[3 * dd:4 * dd]
    c1 = (jax.nn.sigmoid(i1) * jnp.tanh(g1g))[None, :]
    q1 = jax.nn.sigmoid(o1)[None, :] * jnp.tanh(c1)
    return bsum, q1, c1


def _s2s_node(x, wi, wh, bi, bh):
    bsum, q1, c1 = _s2s_consts(wi, wh, bi, bh, 128)
    return pl.pallas_call(
        _s2s_node_body,
        out_shape=jax.ShapeDtypeStruct((1, 256), F32),
    )(x, wi[:128], wi[128:], wh, bsum, q1, c1)


def _s2s_edge(x8, wi, wh, bi, bh, t16, m8, k8, t16t):
    bsum, q1, c1 = _s2s_consts(wi, wh, bi, bh, 16)
    e8 = x8.shape[0]
    chk = 4000
    assert e8 % chk == 0
    return pl.pallas_call(
        functools.partial(_s2s_edge_body, e8 // chk, chk),
        out_shape=jax.ShapeDtypeStruct((1, 32), F32),
        compiler_params=pltpu.CompilerParams(
            vmem_limit_bytes=60 * 1024 * 1024),
    )(x8, wi[:16], wi[16:], wh, bsum, q1, c1, t16, m8, k8, t16t)


def _k6_body(ne, nn, ns_ref, es_ref, gf_ref, sums_ref, ngs_ref, wg_ref,
             bg_ref, d_ref, db_ref, out_ref):
    eg = jnp.sum(sums_ref[...], axis=0, keepdims=True) / ne
    ng = ngs_ref[...] / nn
    wg = wg_ref[...]
    gf = gf_ref[...]
    gf_n = (jnp.dot(eg, wg[0:16], preferred_element_type=F32, precision=HI)
            + jnp.dot(ng, wg[16:144], preferred_element_type=F32,
                      precision=HI)
            + jnp.dot(gf, wg[144:176], preferred_element_type=F32,
                      precision=HI)
            + bg_ref[...])
    gf2 = gf + gf_n
    d = d_ref[...]
    out_ref[...] = (jnp.dot(ns_ref[...], d[0:256], preferred_element_type=F32,
                            precision=HI)
                    + jnp.dot(es_ref[...], d[256:288],
                              preferred_element_type=F32, precision=HI)
                    + jnp.dot(gf2, d[288:320], preferred_element_type=F32,
                              precision=HI)
                    + db_ref[...])


def _k6(ne, nn, ns, es, gf, sums, ngs, wg, bg, d, db):
    return pl.pallas_call(
        functools.partial(_k6_body, float(ne), float(nn)),
        out_shape=jax.ShapeDtypeStruct((1, 1), F32),
    )(ns, es, gf, sums, ngs, wg, bg, d, db)


def kernel(node_features, edge_index, edge_features, global_features,
           node_batch_map, edge_batch_map, params):
    p = params
    nf, ef, gf = node_features, edge_features, global_features
    n, e = nf.shape[0], ef.shape[0]
    src = edge_index[0]
    dst = edge_index[1]

    we = p['ew0'] @ p['ew1'] @ p['ew2'] @ p['edw']
    be = (((p['eb0'] @ p['ew1'] + p['eb1']) @ p['ew2'] + p['eb2']) @ p['edw']
          + p['edb'])[None, :]
    wn = p['nw0'] @ p['nw1'] @ p['nw2'] @ p['ndw']
    bn = (((p['nb0'] @ p['nw1'] + p['nb1']) @ p['nw2'] + p['nb2']) @ p['ndw']
          + p['ndb'])[None, :]
    wg = p['gw0'] @ p['gw1'] @ p['gw2'] @ p['gdw']
    bg = (((p['gb0'] @ p['gw1'] + p['gb1']) @ p['gw2'] + p['gb2']) @ p['gdw']
          + p['gdb'])[None, :]
    d_head = p['d1w'] @ p['d2w'] @ p['ow']
    db_head = ((p['d1b'] @ p['d2w'] + p['d2b']) @ p['ow'] + p['ob'])[None, :]

    eye8 = jnp.eye(8, dtype=F32)
    eye16 = jnp.eye(16, dtype=F32)
    t16 = jnp.kron(jnp.ones((1, 8), F32), eye16)
    m8 = jnp.kron(eye8, jnp.ones((16, 1), F32))
    k8 = m8.T
    t16t = t16.T
    a_blk = jnp.kron(eye8, we[256:272])

    w_all = jnp.concatenate([we[0:128], we[128:256], wn[0:128]], axis=1)
    p_tab, q_tab, nfb = _k1(nf, w_all)

    efa8 = _k2(ef.reshape(e // 8, 128), a_blk, gf, we[272:304], be, t16)

    xe8, acc, degp, sums = _sc_edge(src, dst, ef.reshape(e // 8, 128),
                                    efa8, p_tab, q_tab)
    acc = acc.reshape(NC, n, 16)
    sums = sums.reshape(NW, 16)

    xn, ngs = _k3(nf, nfb, acc, degp, wn[128:144], gf, wn[144:176], bn)

    ns = _s2s_node(xn, p['s2sn_wi'], p['s2sn_wh'], p['s2sn_bi'],
                   p['s2sn_bh'])
    es = _s2s_edge(xe8, p['s2se_wi'], p['s2se_wh'],
                   p['s2se_bi'], p['s2se_bh'], t16, m8, k8, t16t)

    return _k6(e, n, ns, es, gf, sums, ngs, wg, bg, d_head, db_head)


# speedup vs baseline: 1.0838x; 1.0838x over previous
import functools

import jax
import jax.numpy as jnp
from jax.experimental import pallas as pl
from jax.experimental.pallas import tpu as pltpu
from jax.experimental.pallas import tpu_sc as plsc

F32 = jnp.float32
HI = jax.lax.Precision.HIGHEST

NC, NS, L = 2, 16, 16
NW = NC * NS
CH = 128


def _sig(x):
    return 1.0 / (1.0 + jnp.exp(-x))


def _tanh(x):
    a = jnp.abs(x)
    e = jnp.exp(-2.0 * a)
    return jnp.sign(x) * (1.0 - e) / (1.0 + e)


def _k1_body(nf_ref, w_ref, p_ref, q_ref, nfb_ref):
    y = jnp.dot(nf_ref[...], w_ref[...], preferred_element_type=F32,
                precision=HI)
    p_ref[...] = y[:, 0:16]
    q_ref[...] = y[:, 16:32]
    nfb_ref[...] = y[:, 32:160]


def _k1(nf, w_all):
    n = nf.shape[0]
    blk = 2000
    grid = n // blk
    return pl.pallas_call(
        _k1_body,
        grid=(grid,),
        in_specs=[
            pl.BlockSpec((blk, 128), lambda i: (i, 0)),
            pl.BlockSpec((128, 160), lambda i: (0, 0)),
        ],
        out_specs=(
            pl.BlockSpec((blk, 16), lambda i: (i, 0)),
            pl.BlockSpec((blk, 16), lambda i: (i, 0)),
            pl.BlockSpec((blk, 128), lambda i: (i, 0)),
        ),
        out_shape=(
            jax.ShapeDtypeStruct((n, 16), F32),
            jax.ShapeDtypeStruct((n, 16), F32),
            jax.ShapeDtypeStruct((n, 128), F32),
        ),
    )(nf, w_all)


def _k2_body(x_ref, ab_ref, gf_ref, agf_ref, be_ref, t16_ref, out_ref):
    ce = jnp.dot(gf_ref[...], agf_ref[...], preferred_element_type=F32,
                 precision=HI) + be_ref[...]
    ce128 = jnp.dot(ce, t16_ref[...], preferred_element_type=F32,
                    precision=HI)
    out_ref[...] = jnp.dot(x_ref[...], ab_ref[...],
                           preferred_element_type=F32, precision=HI) + ce128


def _k2(ef8, a_blk, gf, a_gf, be, t16):
    e8 = ef8.shape[0]
    blk = 8000
    grid = e8 // blk
    return pl.pallas_call(
        _k2_body,
        grid=(grid,),
        in_specs=[
            pl.BlockSpec((blk, 128), lambda i: (i, 0)),
            pl.BlockSpec((128, 128), lambda i: (0, 0)),
            pl.BlockSpec((1, 32), lambda i: (0, 0)),
            pl.BlockSpec((32, 16), lambda i: (0, 0)),
            pl.BlockSpec((1, 16), lambda i: (0, 0)),
            pl.BlockSpec((16, 128), lambda i: (0, 0)),
        ],
        out_specs=pl.BlockSpec((blk, 128), lambda i: (i, 0)),
        out_shape=jax.ShapeDtypeStruct((e8, 128), F32),
    )(ef8, a_blk, gf, a_gf, be, t16)


def _sc_edge(src, dst, ef8, efa8, p_tab, q_tab):
    e = ef8.shape[0] * 8
    n = p_tab.shape[0]
    nchunks = e // CH
    base_c, extra = nchunks // NW, nchunks % NW
    nrows_t = n // NS
    CR = CH // 8

    mesh = plsc.VectorSubcoreMesh(core_axis_name="c", subcore_axis_name="s")

    @functools.partial(
        pl.kernel,
        out_type=(
            jax.ShapeDtypeStruct((e // 8, 128), F32),
            jax.ShapeDtypeStruct((NC, NS, nrows_t, 16), F32),
            jax.ShapeDtypeStruct((NC, NS, n), F32),
            jax.ShapeDtypeStruct((NW, 1, 16), F32),
        ),
        mesh=mesh,
        compiler_params=pltpu.CompilerParams(needs_layout_passes=False,
                                             use_tc_tiling_on_sc=False),
        scratch_types=dict(
            src_v=pltpu.VMEM((CH,), jnp.int32),
            dst_v=pltpu.VMEM((CH,), jnp.int32),
            ef_v=pltpu.VMEM((CR, 128), F32),
            efa_v=pltpu.VMEM((CR, 128), F32),
            xe_v=pltpu.VMEM((CR, 128), F32),
            pr_v=pltpu.VMEM((CH, 16), F32),
            qr_v=pltpu.VMEM((CH, 16), F32),
            efn_v=pltpu.VMEM((CH, 16), F32),
            deg_v=pltpu.VMEM((n,), F32),
            sum_v=pltpu.VMEM((16,), F32),
            zrow_v=pltpu.VMEM((n // NS, 16), F32),
            acc_sh=pltpu.VMEM_SHARED((n, 16), F32),
            sem_p=pltpu.SemaphoreType.DMA,
            sem_q=pltpu.SemaphoreType.DMA,
        ),
    )
    def k(src_h, dst_h, ef_h, efa_h, p_h, q_h, xe_h, acc_h, deg_h, sums_h,
          src_v, dst_v, ef_v, efa_v, xe_v, pr_v, qr_v, efn_v, deg_v, sum_v,
          zrow_v, acc_sh, sem_p, sem_q):
        cid = jax.lax.axis_index("c")
        sid = jax.lax.axis_index("s")
        wid = sid * NC + cid

        zf = jnp.zeros((L,), F32)

        def zloop(i, _):
            deg_v[pl.ds(i * L, L)] = zf
            return 0
        jax.lax.fori_loop(0, n // L, zloop, 0)

        def zrow(i, _):
            zrow_v[i, :] = zf
            return 0
        jax.lax.fori_loop(0, nrows_t, zrow, 0)

        pltpu.sync_copy(zrow_v, acc_sh.at[pl.ds(sid * nrows_t, nrows_t)])
        plsc.subcore_barrier()

        ones = jnp.ones((L,), F32)
        my_chunks = base_c + jnp.where(wid < extra, 1, 0)
        chunk0 = wid * base_c + jnp.minimum(wid, extra)

        def chunk_body(i, acc_sum):
            c = chunk0 + i
            base = c * CH
            pltpu.sync_copy(src_h.at[pl.ds(base, CH)], src_v)
            pltpu.sync_copy(dst_h.at[pl.ds(base, CH)], dst_v)
            cp = pltpu.async_copy(p_h.at[src_v], pr_v, sem_p)
            cq = pltpu.async_copy(q_h.at[dst_v], qr_v, sem_q)
            pltpu.sync_copy(ef_h.at[pl.ds(c * CR, CR)], ef_v)
            pltpu.sync_copy(efa_h.at[pl.ds(c * CR, CR)], efa_v)
            cp.wait()
            cq.wait()

            def row_body(r, s):
                g = jax.lax.shift_right_logical(r, 3)
                o = jax.lax.bitwise_and(r, 7) * 16
                efn = efa_v[g, pl.ds(o, 16)] + pr_v[r, :] + qr_v[r, :]
                efn_v[r, :] = efn
                xe_v[g, pl.ds(o, 16)] = ef_v[g, pl.ds(o, 16)] + efn
                return s + efn
            acc_sum = jax.lax.fori_loop(0, CH, row_body, acc_sum)

            def deg_body(r, _):
                idx = src_v[pl.ds(r * L, L)]
                plsc.addupdate_scatter(deg_v, [idx], ones)
                return 0
            jax.lax.fori_loop(0, CH // L, deg_body, 0)

            pltpu.sync_copy(efn_v, acc_sh.at[src_v], add=True)
            pltpu.sync_copy(xe_v, xe_h.at[pl.ds(c * CR, CR)])
            return acc_sum

        total = jax.lax.fori_loop(0, my_chunks, chunk_body,
                                  jnp.zeros((16,), F32))
        sum_v[...] = total

        pltpu.sync_copy(sum_v, sums_h.at[wid, 0])
        pltpu.sync_copy(deg_v, deg_h.at[cid, sid])
        plsc.subcore_barrier()
        pltpu.sync_copy(acc_sh.at[pl.ds(sid * nrows_t, nrows_t)],
                        acc_h.at[cid, sid])

    return k(src, dst, ef8, efa8, p_tab, q_tab)


def _k3_body(nf_ref, nfb_ref, acc_ref, deg_ref, bem_ref, gf_ref, bgf_ref,
             bn_ref, xn_ref, ngs_ref):
    acc = acc_ref[0] + acc_ref[1]
    deg = jnp.sum(deg_ref[...], axis=(0, 1))
    emean = acc / jnp.maximum(deg, 1.0)[:, None]
    cn = jnp.dot(gf_ref[...], bgf_ref[...], preferred_element_type=F32,
                 precision=HI) + bn_ref[...]
    nf_n = nfb_ref[...] + jnp.dot(emean, bem_ref[...],
                                  preferred_element_type=F32,
                                  precision=HI) + cn
    xn_ref[...] = nf_ref[...] + nf_n
    ngs_ref[...] = jnp.sum(nf_n, axis=0, keepdims=True)


def _k3(nf, nfb, acc, degp, b_em, gf, b_gf, bn):
    n = nf.shape[0]
    return pl.pallas_call(
        _k3_body,
        out_shape=(
            jax.ShapeDtypeStruct((n, 128), F32),
            jax.ShapeDtypeStruct((1, 128), F32),
        ),
    )(nf, nfb, acc, degp, b_em, gf, b_gf, bn)


def _s2s_lstm(dd, h, r, c, wih, wil, wh, bsum):
    gates = (jnp.dot(h, wih, preferred_element_type=F32, precision=HI)
             + jnp.dot(r, wil, preferred_element_type=F32, precision=HI)
             + jnp.dot(h, wh, preferred_element_type=F32, precision=HI)
             + bsum)
    ig = gates[:, 0:dd]
    fg = gates[:, dd:2 * dd]
    gg = gates[:, 2 * dd:3 * dd]
    og = gates[:, 3 * dd:4 * dd]
    c = _sig(fg) * c + _sig(ig) * _tanh(gg)
    h = _sig(og) * _tanh(c)
    return h, c


def _s2s_node_body(x_ref, wih_ref, wil_ref, wh_ref, bsum_ref, q1_ref, c1_ref,
                   out_ref):
    x = x_ref[...]

    def attend(q):
        e = jnp.sum(x * q, axis=1, keepdims=True)
        m = jnp.max(e)
        a = jnp.exp(e - m)
        s = jnp.sum(a)
        return jnp.sum(a * x, axis=0, keepdims=True) / s

    h = q1_ref[...]
    c = c1_ref[...]
    r = attend(h)
    for _ in range(2):
        h, c = _s2s_lstm(128, h, r, c, wih_ref[...], wil_ref[...],
                         wh_ref[...], bsum_ref[...])
        r = attend(h)
    out_ref[:, 0:128] = h
    out_ref[:, 128:256] = r


def _s2s_edge_body(nchunks, chk, x_ref, wih_ref, wil_ref, wh_ref, bsum_ref,
                   q1_ref, c1_ref, t16_ref, m8_ref, k8_ref, t16t_ref,
                   out_ref):
    m8 = m8_ref[...]
    k8 = k8_ref[...]

    def attend(q):
        tq = jnp.dot(q, t16_ref[...], preferred_element_type=F32,
                     precision=HI)
        m = jnp.float32(-3.4e38)
        s = jnp.float32(0.0)
        r128 = jnp.zeros((1, 128), F32)
        for i in range(nchunks):
            xc = x_ref[pl.ds(i * chk, chk), :]
            eg = jnp.dot(xc * tq, m8, preferred_element_type=F32,
                         precision=HI)
            mn = jnp.maximum(m, jnp.max(eg))
            alpha = jnp.exp(m - mn)
            a = jnp.exp(eg - mn)
            s = s * alpha + jnp.sum(a)
            aw = jnp.dot(a, k8, preferred_element_type=F32, precision=HI)
            r128 = r128 * alpha + jnp.sum(xc * aw, axis=0, keepdims=True)
            m = mn
        return jnp.dot(r128, t16t_ref[...], preferred_element_type=F32,
                       precision=HI) / s

    h = q1_ref[...]
    c = c1_ref[...]
    r = attend(h)
    for _ in range(2):
        h, c = _s2s_lstm(16, h, r, c, wih_ref[...], wil_ref[...],
                         wh_ref[...], bsum_ref[...])
        r = attend(h)
    out_ref[:, 0:16] = h
    out_ref[:, 16:32] = r


def _s2s_consts(wi, wh, bi, bh, dd):
    bsum = (bi + bh)[None, :]
    g1 = bi + bh
    i1 = g1[0:dd]
    g1g = g1[2 * dd:3 * dd]
    o1 = g1[3 * dd:4 * dd]
    c1 = (jax.nn.sigmoid(i1) * jnp.tanh(g1g))[None, :]
    q1 = jax.nn.sigmoid(o1)[None, :] * jnp.tanh(c1)
    return bsum, q1, c1


def _s2s_node(x, wi, wh, bi, bh):
    bsum, q1, c1 = _s2s_consts(wi, wh, bi, bh, 128)
    return pl.pallas_call(
        _s2s_node_body,
        out_shape=jax.ShapeDtypeStruct((1, 256), F32),
    )(x, wi[:128], wi[128:], wh, bsum, q1, c1)


def _s2s_edge(x8, wi, wh, bi, bh, t16, m8, k8, t16t):
    bsum, q1, c1 = _s2s_consts(wi, wh, bi, bh, 16)
    e8 = x8.shape[0]
    chk = 4000
    assert e8 % chk == 0
    return pl.pallas_call(
        functools.partial(_s2s_edge_body, e8 // chk, chk),
        out_shape=jax.ShapeDtypeStruct((1, 32), F32),
        compiler_params=pltpu.CompilerParams(
            vmem_limit_bytes=60 * 1024 * 1024),
    )(x8, wi[:16], wi[16:], wh, bsum, q1, c1, t16, m8, k8, t16t)


def _k6_body(ne, nn, ns_ref, es_ref, gf_ref, sums_ref, ngs_ref, wg_ref,
             bg_ref, d_ref, db_ref, out_ref):
    eg = jnp.sum(sums_ref[...], axis=0, keepdims=True) / ne
    ng = ngs_ref[...] / nn
    wg = wg_ref[...]
    gf = gf_ref[...]
    gf_n = (jnp.dot(eg, wg[0:16], preferred_element_type=F32, precision=HI)
            + jnp.dot(ng, wg[16:144], preferred_element_type=F32,
                      precision=HI)
            + jnp.dot(gf, wg[144:176], preferred_element_type=F32,
                      precision=HI)
            + bg_ref[...])
    gf2 = gf + gf_n
    d = d_ref[...]
    out_ref[...] = (jnp.dot(ns_ref[...], d[0:256], preferred_element_type=F32,
                            precision=HI)
                    + jnp.dot(es_ref[...], d[256:288],
                              preferred_element_type=F32, precision=HI)
                    + jnp.dot(gf2, d[288:320], preferred_element_type=F32,
                              precision=HI)
                    + db_ref[...])


def _k6(ne, nn, ns, es, gf, sums, ngs, wg, bg, d, db):
    return pl.pallas_call(
        functools.partial(_k6_body, float(ne), float(nn)),
        out_shape=jax.ShapeDtypeStruct((1, 1), F32),
    )(ns, es, gf, sums, ngs, wg, bg, d, db)


def kernel(node_features, edge_index, edge_features, global_features,
           node_batch_map, edge_batch_map, params):
    p = params
    nf, ef, gf = node_features, edge_features, global_features
    n, e = nf.shape[0], ef.shape[0]
    src = edge_index[0]
    dst = edge_index[1]

    we = p['ew0'] @ p['ew1'] @ p['ew2'] @ p['edw']
    be = (((p['eb0'] @ p['ew1'] + p['eb1']) @ p['ew2'] + p['eb2']) @ p['edw']
          + p['edb'])[None, :]
    wn = p['nw0'] @ p['nw1'] @ p['nw2'] @ p['ndw']
    bn = (((p['nb0'] @ p['nw1'] + p['nb1']) @ p['nw2'] + p['nb2']) @ p['ndw']
          + p['ndb'])[None, :]
    wg = p['gw0'] @ p['gw1'] @ p['gw2'] @ p['gdw']
    bg = (((p['gb0'] @ p['gw1'] + p['gb1']) @ p['gw2'] + p['gb2']) @ p['gdw']
          + p['gdb'])[None, :]
    d_head = p['d1w'] @ p['d2w'] @ p['ow']
    db_head = ((p['d1b'] @ p['d2w'] + p['d2b']) @ p['ow'] + p['ob'])[None, :]

    eye8 = jnp.eye(8, dtype=F32)
    eye16 = jnp.eye(16, dtype=F32)
    t16 = jnp.kron(jnp.ones((1, 8), F32), eye16)
    m8 = jnp.kron(eye8, jnp.ones((16, 1), F32))
    k8 = m8.T
    t16t = t16.T
    a_blk = jnp.kron(eye8, we[256:272])

    w_all = jnp.concatenate([we[0:128], we[128:256], wn[0:128]], axis=1)
    p_tab, q_tab, nfb = _k1(nf, w_all)

    efa8 = _k2(ef.reshape(e // 8, 128), a_blk, gf, we[272:304], be, t16)

    xe8, acc, degp, sums = _sc_edge(src, dst, ef.reshape(e // 8, 128),
                                    efa8, p_tab, q_tab)
    acc = acc.reshape(NC, n, 16)
    sums = sums.reshape(NW, 16)

    xn, ngs = _k3(nf, nfb, acc, degp, wn[128:144], gf, wn[144:176], bn)

    ns = _s2s_node(xn, p['s2sn_wi'], p['s2sn_wh'], p['s2sn_bi'],
                   p['s2sn_bh'])
    es = _s2s_edge(xe8, p['s2se_wi'], p['s2se_wh'],
                   p['s2se_bi'], p['s2se_bh'], t16, m8, k8, t16t)

    return _k6(e, n, ns, es, gf, sums, ngs, wg, bg, d_head, db_head)
